# baseline (speedup 1.0000x reference)
import functools

import jax
import jax.numpy as jnp
from jax import lax
from jax.experimental import pallas as pl
from jax.experimental.pallas import tpu as pltpu
from jax.experimental.pallas import tpu_sc as plsc

MASK_IDX = 32
ZERO_IDX = 33
D = 1024
LANES = 16
NUM_WORKERS = 32
ROWS_PER_GATHER = 64


@functools.lru_cache(maxsize=None)
def _build_sc_kernel(n_tokens: int):
    per_w = n_tokens // NUM_WORKERS
    n_sub = per_w // ROWS_PER_GATHER
    mesh = plsc.VectorSubcoreMesh(core_axis_name="c", subcore_axis_name="s")

    @functools.partial(
        pl.kernel,
        mesh=mesh,
        out_type=jax.ShapeDtypeStruct((n_tokens, D), jnp.float32),
        scratch_types=[
            pltpu.VMEM((per_w,), jnp.int32),
            pltpu.VMEM((per_w,), jnp.int32),
            pltpu.VMEM((per_w,), jnp.int32),
            pltpu.VMEM((per_w,), jnp.int32),
            pltpu.VMEM((ROWS_PER_GATHER, D), jnp.float32),
            pltpu.SemaphoreType.DMA,
        ],
    )
    def sc_embed(table_hbm, tok_hbm, aa_hbm, pad_hbm, out_hbm,
                 tok_v, aa_v, pad_v, idx_v, rows_v, sem):
        wid = lax.axis_index("s") * 2 + lax.axis_index("c")
        base = wid * per_w
        pltpu.sync_copy(tok_hbm.at[pl.ds(base, per_w)], tok_v)
        pltpu.sync_copy(aa_hbm.at[pl.ds(base, per_w)], aa_v)
        pltpu.sync_copy(pad_hbm.at[pl.ds(base, per_w)], pad_v)

        def idx_body(i, carry):
            sl = pl.ds(pl.multiple_of(i * LANES, LANES), LANES)
            t = tok_v[sl]
            a = aa_v[sl]
            p = pad_v[sl]
            idx = jnp.where(a != 0, MASK_IDX, t)
            idx_v[sl] = jnp.where(p != 0, ZERO_IDX, idx)
            return carry

        lax.fori_loop(0, per_w // LANES, idx_body, 0)

        def gather_body(j, carry):
            off = pl.multiple_of(j * ROWS_PER_GATHER, ROWS_PER_GATHER)
            pltpu.async_copy(
                table_hbm.at[idx_v.at[pl.ds(off, ROWS_PER_GATHER)]],
                rows_v, sem).wait()
            pltpu.sync_copy(rows_v, out_hbm.at[pl.ds(base + off, ROWS_PER_GATHER)])
            return carry

        lax.fori_loop(0, n_sub, gather_body, 0)

    return sc_embed


def kernel(tokens, padding_mask, mask_aa, table):
    B, L = tokens.shape
    tok = tokens.reshape(-1).astype(jnp.int32)
    aa = mask_aa.reshape(-1).astype(jnp.int32)
    pad = padding_mask.reshape(-1).astype(jnp.int32)
    table_padded = jnp.concatenate(
        [table, jnp.zeros((1, table.shape[1]), table.dtype)], axis=0)
    out = _build_sc_kernel(B * L)(table_padded, tok, aa, pad)
    return out.reshape(B, L, D)
